# Initial kernel scaffold; baseline (speedup 1.0000x reference)
import jax
import jax.numpy as jnp
from jax.experimental import pallas as pl


def kernel(x_e, pos_e, edge_index_e, edge_attr_e, batch_node, batch_edge, W1, b1, W2, b2):
    raise NotImplementedError("write your pallas kernel here")


# speedup vs baseline: 6.4895x; 6.4895x over previous
import functools

import jax
import jax.numpy as jnp
from jax.experimental import pallas as pl
from jax.experimental.pallas import tpu as pltpu

N_ROWS = 10000
HIDDEN = 256
NUM_SEGS = 64
NUM_CLASSES = 10
BLK = 400
NBLK = N_ROWS // BLK


def _pool_mlp_kernel(ids_ref, x_ref, w1_ref, b1_ref, w2_ref, b2_ref,
                     out_ref, sums_ref, cnts_ref):
    step = pl.program_id(0)

    @pl.when(step == 0)
    def _init():
        sums_ref[...] = jnp.zeros_like(sums_ref)
        cnts_ref[...] = jnp.zeros_like(cnts_ref)

    ids = ids_ref[0]
    seg_iota = jax.lax.broadcasted_iota(jnp.int32, (NUM_SEGS, BLK), 0)
    onehot = (ids == seg_iota).astype(jnp.float32)
    sums_ref[...] += jax.lax.dot(
        onehot, x_ref[...],
        precision=jax.lax.Precision.HIGHEST,
        preferred_element_type=jnp.float32)
    cnts_ref[...] += jnp.sum(onehot, axis=1, keepdims=True)

    @pl.when(step == NBLK - 1)
    def _head():
        pool = sums_ref[...] / jnp.maximum(cnts_ref[...], 1.0)
        h = jax.lax.dot(pool, w1_ref[...],
                        precision=jax.lax.Precision.HIGHEST,
                        preferred_element_type=jnp.float32)
        h = jnp.maximum(h + b1_ref[...], 0.0)
        logits = jax.lax.dot(h, w2_ref[...],
                             precision=jax.lax.Precision.HIGHEST,
                             preferred_element_type=jnp.float32)
        out_ref[...] = logits + b2_ref[...]


@functools.partial(jax.jit, static_argnames=())
def _run(x_e, batch_node, W1, b1, W2, b2):
    ids3 = batch_node.astype(jnp.int32).reshape(NBLK, 1, BLK)
    b1r = b1.reshape(1, HIDDEN // 2)
    b2r = b2.reshape(1, NUM_CLASSES)
    return pl.pallas_call(
        _pool_mlp_kernel,
        grid=(NBLK,),
        in_specs=[
            pl.BlockSpec((1, 1, BLK), lambda i: (i, 0, 0)),
            pl.BlockSpec((BLK, HIDDEN), lambda i: (i, 0)),
            pl.BlockSpec((HIDDEN, HIDDEN // 2), lambda i: (0, 0)),
            pl.BlockSpec((1, HIDDEN // 2), lambda i: (0, 0)),
            pl.BlockSpec((HIDDEN // 2, NUM_CLASSES), lambda i: (0, 0)),
            pl.BlockSpec((1, NUM_CLASSES), lambda i: (0, 0)),
        ],
        out_specs=pl.BlockSpec((NUM_SEGS, NUM_CLASSES), lambda i: (0, 0)),
        out_shape=jax.ShapeDtypeStruct((NUM_SEGS, NUM_CLASSES), jnp.float32),
        scratch_shapes=[
            pltpu.VMEM((NUM_SEGS, HIDDEN), jnp.float32),
            pltpu.VMEM((NUM_SEGS, 1), jnp.float32),
        ],
    )(ids3, x_e, W1, b1r, W2, b2r)


def kernel(x_e, pos_e, edge_index_e, edge_attr_e, batch_node, batch_edge,
           W1, b1, W2, b2):
    return _run(x_e, batch_node, W1, b1, W2, b2)


# speedup vs baseline: 10.7664x; 1.6590x over previous
import functools

import jax
import jax.numpy as jnp
from jax.experimental import pallas as pl
from jax.experimental.pallas import tpu as pltpu

N_ROWS = 10000
HIDDEN = 256
NUM_SEGS = 64
NUM_CLASSES = 10
BLK = 1000
NBLK = N_ROWS // BLK


def _pool_mlp_kernel(ids_ref, x_ref, w1_ref, b1_ref, w2_ref, b2_ref,
                     out_ref, sums_ref, cnts_ref):
    step = pl.program_id(0)

    @pl.when(step == 0)
    def _init():
        sums_ref[...] = jnp.zeros_like(sums_ref)
        cnts_ref[...] = jnp.zeros_like(cnts_ref)

    ids = ids_ref[0]
    seg_iota = jax.lax.broadcasted_iota(jnp.int32, (NUM_SEGS, BLK), 0)
    onehot = (ids == seg_iota).astype(jnp.float32)
    oh16 = onehot.astype(jnp.bfloat16)
    x = x_ref[...]
    x_hi = x.astype(jnp.bfloat16)
    x_lo = (x - x_hi.astype(jnp.float32)).astype(jnp.bfloat16)
    sums_ref[...] += (
        jax.lax.dot(oh16, x_hi, preferred_element_type=jnp.float32)
        + jax.lax.dot(oh16, x_lo, preferred_element_type=jnp.float32))
    cnts_ref[...] += jnp.sum(onehot, axis=1, keepdims=True)

    @pl.when(step == NBLK - 1)
    def _head():
        pool = sums_ref[...] / jnp.maximum(cnts_ref[...], 1.0)
        h = jax.lax.dot(pool, w1_ref[...],
                        precision=jax.lax.Precision.HIGHEST,
                        preferred_element_type=jnp.float32)
        h = jnp.maximum(h + b1_ref[...], 0.0)
        logits = jax.lax.dot(h, w2_ref[...],
                             precision=jax.lax.Precision.HIGHEST,
                             preferred_element_type=jnp.float32)
        out_ref[...] = logits + b2_ref[...]


@functools.partial(jax.jit, static_argnames=())
def _run(x_e, batch_node, W1, b1, W2, b2):
    ids3 = batch_node.astype(jnp.int32).reshape(NBLK, 1, BLK)
    b1r = b1.reshape(1, HIDDEN // 2)
    b2r = b2.reshape(1, NUM_CLASSES)
    return pl.pallas_call(
        _pool_mlp_kernel,
        grid=(NBLK,),
        in_specs=[
            pl.BlockSpec((1, 1, BLK), lambda i: (i, 0, 0)),
            pl.BlockSpec((BLK, HIDDEN), lambda i: (i, 0)),
            pl.BlockSpec((HIDDEN, HIDDEN // 2), lambda i: (0, 0)),
            pl.BlockSpec((1, HIDDEN // 2), lambda i: (0, 0)),
            pl.BlockSpec((HIDDEN // 2, NUM_CLASSES), lambda i: (0, 0)),
            pl.BlockSpec((1, NUM_CLASSES), lambda i: (0, 0)),
        ],
        out_specs=pl.BlockSpec((NUM_SEGS, NUM_CLASSES), lambda i: (0, 0)),
        out_shape=jax.ShapeDtypeStruct((NUM_SEGS, NUM_CLASSES), jnp.float32),
        scratch_shapes=[
            pltpu.VMEM((NUM_SEGS, HIDDEN), jnp.float32),
            pltpu.VMEM((NUM_SEGS, 1), jnp.float32),
        ],
    )(ids3, x_e, W1, b1r, W2, b2r)


def kernel(x_e, pos_e, edge_index_e, edge_attr_e, batch_node, batch_edge,
           W1, b1, W2, b2):
    return _run(x_e, batch_node, W1, b1, W2, b2)
